# baseline (speedup 1.0000x reference)
import dataclasses
import functools

import jax
import jax.numpy as jnp
from jax import lax
from jax.experimental import pallas as pl
from jax.experimental.pallas import tpu as pltpu
from jax.experimental.pallas import tpu_sc as plsc

N = 10000
E = 320000
H = 128

NC = 2
NS = 16
NW = NC * NS
EPT = E // NW
C = 80
NCH = EPT // C
RPT = 624
TAIL = N - NS * RPT

C2 = 128
NCH2 = 79
EPT2 = NCH2 * C2
EPAD = NW * EPT2
ACCR = N + 8

_MESH = plsc.VectorSubcoreMesh(
    core_axis_name="c", subcore_axis_name="s", num_cores=NC, num_subcores=NS
)

_SC_PARAMS = pltpu.CompilerParams()
if "needs_layout_passes" in pltpu.CompilerParams.__dataclass_fields__:
    _SC_PARAMS = dataclasses.replace(_SC_PARAMS, needs_layout_passes=False)


@functools.partial(
    pl.kernel,
    out_type=jax.ShapeDtypeStruct((NW, 2, N), jnp.float32),
    mesh=_MESH,
    scratch_types=[
        pltpu.VMEM((N,), jnp.float32),
        pltpu.VMEM((N,), jnp.float32),
        pltpu.VMEM((NCH, C), jnp.int32),
        pltpu.VMEM((NCH, C), jnp.int32),
    ],
    compiler_params=_SC_PARAMS,
)
def _deg_kernel(ei_hbm, out_hbm, dout_v, din_v, si_v, di_v):
    wid = lax.axis_index("s") * NC + lax.axis_index("c")

    pltpu.sync_copy(ei_hbm.at[0, wid], si_v)
    pltpu.sync_copy(ei_hbm.at[1, wid], di_v)

    @pl.loop(0, N, step=16)
    def _(i):
        z = jnp.zeros((16,), jnp.float32)
        dout_v[pl.ds(i, 16)] = z
        din_v[pl.ds(i, 16)] = z

    @pl.loop(0, NCH)
    def _(k):
        @pl.loop(0, C, step=16)
        def _(j):
            ones = jnp.ones((16,), jnp.float32)
            plsc.addupdate_scatter(dout_v, [si_v[k, pl.ds(j, 16)]], ones)
            plsc.addupdate_scatter(din_v, [di_v[k, pl.ds(j, 16)]], ones)

    pltpu.sync_copy(dout_v, out_hbm.at[wid, 0])
    pltpu.sync_copy(din_v, out_hbm.at[wid, 1])


@functools.partial(
    pl.kernel,
    out_type=jax.ShapeDtypeStruct((NW, N), jnp.float32),
    mesh=_MESH,
    scratch_types=[
        pltpu.VMEM((N,), jnp.float32),
        pltpu.VMEM((N,), jnp.float32),
        pltpu.VMEM((NCH, C), jnp.int32),
        pltpu.VMEM((NCH, C), jnp.int32),
    ],
    compiler_params=_SC_PARAMS,
)
def _cvec_kernel(ei_hbm, norms_hbm, out_hbm, nin_v, c_v, si_v, di_v):
    wid = lax.axis_index("s") * NC + lax.axis_index("c")
    pltpu.sync_copy(norms_hbm.at[1], nin_v)
    pltpu.sync_copy(ei_hbm.at[0, wid], si_v)
    pltpu.sync_copy(ei_hbm.at[1, wid], di_v)

    @pl.loop(0, N, step=16)
    def _(i):
        c_v[pl.ds(i, 16)] = jnp.zeros((16,), jnp.float32)

    @pl.loop(0, NCH)
    def _(k):
        @pl.loop(0, C, step=16)
        def _(j):
            vals = plsc.load_gather(nin_v, [di_v[k, pl.ds(j, 16)]])
            plsc.addupdate_scatter(c_v, [si_v[k, pl.ds(j, 16)]], vals)

    pltpu.sync_copy(c_v, out_hbm.at[wid])


@functools.partial(
    pl.kernel,
    out_type=jax.ShapeDtypeStruct((NC, N, H), jnp.float32),
    mesh=_MESH,
    scratch_types=[
        pltpu.VMEM_SHARED((ACCR, H), jnp.float32),
        pltpu.VMEM((EPT2,), jnp.int32),
        pltpu.VMEM((1, C2), jnp.int32),
        pltpu.VMEM((1, C2), jnp.int32),
        pltpu.VMEM((C2, H), jnp.float32),
        pltpu.VMEM((C2, H), jnp.float32),
        pltpu.SemaphoreType.DMA,
        pltpu.SemaphoreType.DMA,
        pltpu.SemaphoreType.DMA,
        pltpu.SemaphoreType.DMA,
    ],
)
def _agg_kernel(h_hbm, eis_hbm, eid_hbm, out_hbm, acc_s, si_v, di_a, di_b,
                rows_a, rows_b, sem_ga, sem_gb, sem_ia, sem_ib):
    cid = lax.axis_index("c")
    sid = lax.axis_index("s")
    wid = sid * NC + cid

    pltpu.async_copy(eis_hbm.at[wid], si_v, sem_ga)

    @pl.loop(0, C2)
    def _(r):
        @pl.loop(0, H, step=16)
        def _(h):
            rows_b[r, pl.ds(h, 16)] = jnp.zeros((16,), jnp.float32)

    @pl.loop(0, 4 * C2, step=C2)
    def _(z):
        pltpu.sync_copy(rows_b, acc_s.at[pl.ds(sid * RPT + z, C2)])

    pltpu.sync_copy(
        rows_b.at[pl.ds(0, RPT - 4 * C2)],
        acc_s.at[pl.ds(sid * RPT + 4 * C2, RPT - 4 * C2)],
    )

    @pl.when(sid == 0)
    def _():
        pltpu.sync_copy(rows_b.at[pl.ds(0, 8)], acc_s.at[pl.ds(N, 8)])

    @pl.when(sid == NS - 1)
    def _():
        pltpu.sync_copy(rows_b.at[pl.ds(0, TAIL)], acc_s.at[pl.ds(NS * RPT, TAIL)])

    pltpu.make_async_copy(eis_hbm.at[wid], si_v, sem_ga).wait()
    pltpu.async_copy(h_hbm.at[si_v.at[pl.ds(0, C2)]], rows_a, sem_ga)
    pltpu.async_copy(eid_hbm.at[wid, 0], di_a, sem_ia)

    plsc.subcore_barrier()

    @pl.loop(0, NCH2 - 1, step=2)
    def _(k):
        pltpu.async_copy(h_hbm.at[si_v.at[pl.ds((k + 1) * C2, C2)]], rows_b, sem_gb)
        pltpu.async_copy(eid_hbm.at[wid, k + 1], di_b, sem_ib)
        pltpu.make_async_copy(h_hbm.at[si_v.at[pl.ds(k * C2, C2)]], rows_a, sem_ga).wait()
        pltpu.make_async_copy(eid_hbm.at[wid, k], di_a, sem_ia).wait()
        pltpu.sync_copy(rows_a, acc_s.at[di_a.at[0]], add=True)
        pltpu.async_copy(h_hbm.at[si_v.at[pl.ds((k + 2) * C2, C2)]], rows_a, sem_ga)
        pltpu.async_copy(eid_hbm.at[wid, k + 2], di_a, sem_ia)
        pltpu.make_async_copy(h_hbm.at[si_v.at[pl.ds((k + 1) * C2, C2)]], rows_b, sem_gb).wait()
        pltpu.make_async_copy(eid_hbm.at[wid, k + 1], di_b, sem_ib).wait()
        pltpu.sync_copy(rows_b, acc_s.at[di_b.at[0]], add=True)

    pltpu.make_async_copy(h_hbm.at[si_v.at[pl.ds((NCH2 - 1) * C2, C2)]], rows_a, sem_ga).wait()
    pltpu.make_async_copy(eid_hbm.at[wid, NCH2 - 1], di_a, sem_ia).wait()
    pltpu.sync_copy(rows_a, acc_s.at[di_a.at[0]], add=True)

    plsc.subcore_barrier()
    pltpu.sync_copy(
        acc_s.at[pl.ds(sid * RPT, RPT)], out_hbm.at[cid, pl.ds(sid * RPT, RPT)]
    )

    @pl.when(sid == NS - 1)
    def _():
        pltpu.sync_copy(
            acc_s.at[pl.ds(NS * RPT, TAIL)], out_hbm.at[cid, pl.ds(NS * RPT, TAIL)]
        )


def _prep_body(deg_ref, x_ref, w1_ref, norms_ref, h1_ref):
    deg = jnp.sum(deg_ref[...], axis=0)
    norms = jnp.where(deg > 0, lax.rsqrt(jnp.maximum(deg, 1.0)), 0.0)
    norms_ref[...] = norms
    h = x_ref[...] * norms[0][:, None]
    h1_ref[...] = jnp.dot(h, w1_ref[...], preferred_element_type=jnp.float32)


def _epi_body(agg_ref, norms_ref, b_ref, w_ref, h_ref):
    ni = norms_ref[1][:, None]
    x = jnp.maximum(
        (agg_ref[0] + agg_ref[1]) * ni + b_ref[...][None, :], 0.0
    )
    h_ref[...] = jnp.dot(
        x * norms_ref[0][:, None], w_ref[...], preferred_element_type=jnp.float32
    )


def _fin_body(agg_ref, norms_ref, b2_ref, cp_ref, w3_ref, b3_ref, out_ref):
    ni = norms_ref[1][:, None]
    x3 = jnp.maximum(
        (agg_ref[0] + agg_ref[1]) * ni + b2_ref[...][None, :], 0.0
    )
    c = jnp.sum(cp_ref[...], axis=0)
    cn = (c * norms_ref[0])[None, :]
    t = jnp.dot(cn, x3, preferred_element_type=jnp.float32)
    out_ref[...] = (
        jnp.dot(t, w3_ref[...], preferred_element_type=jnp.float32) / N
        + b3_ref[...][None, :]
    )


def kernel(features, edge_index, W1, b1, W2, b2, W3, b3):
    ei = edge_index.reshape(2, NW, NCH, C)
    eis = jnp.concatenate(
        [edge_index[0], jnp.zeros((EPAD - E,), jnp.int32)]
    ).reshape(NW, EPT2)
    eid = jnp.concatenate(
        [edge_index[1], jnp.full((EPAD - E,), N, jnp.int32)]
    ).reshape(NW, NCH2, 1, C2)

    degp = _deg_kernel(ei)

    norms, h1 = pl.pallas_call(
        _prep_body,
        out_shape=[
            jax.ShapeDtypeStruct((2, N), jnp.float32),
            jax.ShapeDtypeStruct((N, H), jnp.float32),
        ],
    )(degp, features, W1)

    cpart = _cvec_kernel(ei, norms)

    agg1 = _agg_kernel(h1, eis, eid)

    h2 = pl.pallas_call(
        _epi_body,
        out_shape=jax.ShapeDtypeStruct((N, H), jnp.float32),
    )(agg1, norms, b1, W2)

    agg2 = _agg_kernel(h2, eis, eid)

    pooled = pl.pallas_call(
        _fin_body,
        out_shape=jax.ShapeDtypeStruct((1, 1), jnp.float32),
    )(agg2, norms, b2, cpart, W3, b3)

    return pooled


# speedup vs baseline: 1.1345x; 1.1345x over previous
import dataclasses
import functools

import jax
import jax.numpy as jnp
from jax import lax
from jax.experimental import pallas as pl
from jax.experimental.pallas import tpu as pltpu
from jax.experimental.pallas import tpu_sc as plsc

N = 10000
E = 320000
H = 128

NC = 2
NS = 16
NW = NC * NS
EPT = E // NW
C = 80
NCH = EPT // C
RPT = 624
TAIL = N - NS * RPT

C2 = 128
NCH2 = 79
EPT2 = NCH2 * C2
EPAD = NW * EPT2
NP = N + 8

_MESH = plsc.VectorSubcoreMesh(
    core_axis_name="c", subcore_axis_name="s", num_cores=NC, num_subcores=NS
)

_SC_PARAMS = pltpu.CompilerParams()
if "needs_layout_passes" in pltpu.CompilerParams.__dataclass_fields__:
    _SC_PARAMS = dataclasses.replace(_SC_PARAMS, needs_layout_passes=False)


@functools.partial(
    pl.kernel,
    out_type=jax.ShapeDtypeStruct((NW, 2, N), jnp.float32),
    mesh=_MESH,
    scratch_types=[
        pltpu.VMEM((N,), jnp.float32),
        pltpu.VMEM((N,), jnp.float32),
        pltpu.VMEM((NCH, C), jnp.int32),
        pltpu.VMEM((NCH, C), jnp.int32),
    ],
    compiler_params=_SC_PARAMS,
)
def _deg_kernel(ei_hbm, out_hbm, dout_v, din_v, si_v, di_v):
    wid = lax.axis_index("s") * NC + lax.axis_index("c")

    pltpu.sync_copy(ei_hbm.at[0, wid], si_v)
    pltpu.sync_copy(ei_hbm.at[1, wid], di_v)

    @pl.loop(0, N, step=16)
    def _(i):
        z = jnp.zeros((16,), jnp.float32)
        dout_v[pl.ds(i, 16)] = z
        din_v[pl.ds(i, 16)] = z

    @pl.loop(0, NCH)
    def _(k):
        @pl.loop(0, C, step=16)
        def _(j):
            ones = jnp.ones((16,), jnp.float32)
            plsc.addupdate_scatter(dout_v, [si_v[k, pl.ds(j, 16)]], ones)
            plsc.addupdate_scatter(din_v, [di_v[k, pl.ds(j, 16)]], ones)

    pltpu.sync_copy(dout_v, out_hbm.at[wid, 0])
    pltpu.sync_copy(din_v, out_hbm.at[wid, 1])


@functools.partial(
    pl.kernel,
    out_type=jax.ShapeDtypeStruct((NW, N), jnp.float32),
    mesh=_MESH,
    scratch_types=[
        pltpu.VMEM((N,), jnp.float32),
        pltpu.VMEM((N,), jnp.float32),
        pltpu.VMEM((NCH, C), jnp.int32),
        pltpu.VMEM((NCH, C), jnp.int32),
    ],
    compiler_params=_SC_PARAMS,
)
def _cvec_kernel(ei_hbm, norms_hbm, out_hbm, nin_v, c_v, si_v, di_v):
    wid = lax.axis_index("s") * NC + lax.axis_index("c")
    pltpu.sync_copy(norms_hbm.at[1], nin_v)
    pltpu.sync_copy(ei_hbm.at[0, wid], si_v)
    pltpu.sync_copy(ei_hbm.at[1, wid], di_v)

    @pl.loop(0, N, step=16)
    def _(i):
        c_v[pl.ds(i, 16)] = jnp.zeros((16,), jnp.float32)

    @pl.loop(0, NCH)
    def _(k):
        @pl.loop(0, C, step=16)
        def _(j):
            vals = plsc.load_gather(nin_v, [di_v[k, pl.ds(j, 16)]])
            plsc.addupdate_scatter(c_v, [si_v[k, pl.ds(j, 16)]], vals)

    pltpu.sync_copy(c_v, out_hbm.at[wid])


@functools.partial(
    pl.kernel,
    out_type=jax.ShapeDtypeStruct((NC, N, H), jnp.float32),
    mesh=_MESH,
    scratch_types=[
        pltpu.VMEM_SHARED((N, H), jnp.float32),
        pltpu.VMEM((EPT2,), jnp.int32),
        pltpu.VMEM((1, C2), jnp.int32),
        pltpu.VMEM((1, C2), jnp.int32),
        pltpu.VMEM((C2, H), jnp.float32),
        pltpu.VMEM((C2, H), jnp.float32),
        pltpu.SemaphoreType.DMA,
        pltpu.SemaphoreType.DMA,
        pltpu.SemaphoreType.DMA,
        pltpu.SemaphoreType.DMA,
    ],
)
def _agg_kernel(h_hbm, eis_hbm, eid_hbm, out_hbm, acc_s, si_v, di_a, di_b,
                rows_a, rows_b, sem_ga, sem_gb, sem_ia, sem_ib):
    cid = lax.axis_index("c")
    sid = lax.axis_index("s")
    wid = sid * NC + cid

    pltpu.async_copy(eis_hbm.at[wid], si_v, sem_ga)

    @pl.loop(0, C2)
    def _(r):
        @pl.loop(0, H, step=16)
        def _(h):
            rows_b[r, pl.ds(h, 16)] = jnp.zeros((16,), jnp.float32)

    @pl.loop(0, 4 * C2, step=C2)
    def _(z):
        pltpu.sync_copy(rows_b, acc_s.at[pl.ds(sid * RPT + z, C2)])

    pltpu.sync_copy(
        rows_b.at[pl.ds(0, RPT - 4 * C2)],
        acc_s.at[pl.ds(sid * RPT + 4 * C2, RPT - 4 * C2)],
    )

    @pl.when(sid == NS - 1)
    def _():
        pltpu.sync_copy(rows_b.at[pl.ds(0, TAIL)], acc_s.at[pl.ds(NS * RPT, TAIL)])

    pltpu.make_async_copy(eis_hbm.at[wid], si_v, sem_ga).wait()
    pltpu.async_copy(h_hbm.at[si_v.at[pl.ds(0, C2)]], rows_a, sem_ga)
    pltpu.async_copy(eid_hbm.at[wid, 0], di_a, sem_ia)

    plsc.subcore_barrier()

    @pl.loop(0, NCH2 - 1, step=2)
    def _(k):
        pltpu.async_copy(h_hbm.at[si_v.at[pl.ds((k + 1) * C2, C2)]], rows_b, sem_gb)
        pltpu.async_copy(eid_hbm.at[wid, k + 1], di_b, sem_ib)
        pltpu.make_async_copy(h_hbm.at[si_v.at[pl.ds(k * C2, C2)]], rows_a, sem_ga).wait()
        pltpu.make_async_copy(eid_hbm.at[wid, k], di_a, sem_ia).wait()
        pltpu.sync_copy(rows_a, acc_s.at[di_a.at[0]], add=True)
        pltpu.async_copy(h_hbm.at[si_v.at[pl.ds((k + 2) * C2, C2)]], rows_a, sem_ga)
        pltpu.async_copy(eid_hbm.at[wid, k + 2], di_a, sem_ia)
        pltpu.make_async_copy(h_hbm.at[si_v.at[pl.ds((k + 1) * C2, C2)]], rows_b, sem_gb).wait()
        pltpu.make_async_copy(eid_hbm.at[wid, k + 1], di_b, sem_ib).wait()
        pltpu.sync_copy(rows_b, acc_s.at[di_b.at[0]], add=True)

    pltpu.make_async_copy(h_hbm.at[si_v.at[pl.ds((NCH2 - 1) * C2, C2)]], rows_a, sem_ga).wait()
    pltpu.make_async_copy(eid_hbm.at[wid, NCH2 - 1], di_a, sem_ia).wait()
    pltpu.sync_copy(rows_a, acc_s.at[di_a.at[0]], add=True)

    plsc.subcore_barrier()
    pltpu.sync_copy(
        acc_s.at[pl.ds(sid * RPT, RPT)], out_hbm.at[cid, pl.ds(sid * RPT, RPT)]
    )

    @pl.when(sid == NS - 1)
    def _():
        pltpu.sync_copy(
            acc_s.at[pl.ds(NS * RPT, TAIL)], out_hbm.at[cid, pl.ds(NS * RPT, TAIL)]
        )


def _prep_body(deg_ref, x_ref, w1_ref, norms_ref, h1_ref):
    deg = jnp.sum(deg_ref[...], axis=0)
    norms = jnp.where(deg > 0, lax.rsqrt(jnp.maximum(deg, 1.0)), 0.0)
    norms_ref[...] = norms
    h = x_ref[...] * norms[0][:, None]
    h1_ref[:N] = jnp.dot(h, w1_ref[...], preferred_element_type=jnp.float32)
    h1_ref[N:] = jnp.zeros((NP - N, H), jnp.float32)


def _epi_body(agg_ref, norms_ref, b_ref, w_ref, h_ref):
    ni = norms_ref[1][:, None]
    x = jnp.maximum(
        (agg_ref[0] + agg_ref[1]) * ni + b_ref[...][None, :], 0.0
    )
    h_ref[:N] = jnp.dot(
        x * norms_ref[0][:, None], w_ref[...], preferred_element_type=jnp.float32
    )
    h_ref[N:] = jnp.zeros((NP - N, H), jnp.float32)


def _fin_body(agg_ref, norms_ref, b2_ref, cp_ref, w3_ref, b3_ref, out_ref):
    ni = norms_ref[1][:, None]
    x3 = jnp.maximum(
        (agg_ref[0] + agg_ref[1]) * ni + b2_ref[...][None, :], 0.0
    )
    c = jnp.sum(cp_ref[...], axis=0)
    cn = (c * norms_ref[0])[None, :]
    t = jnp.dot(cn, x3, preferred_element_type=jnp.float32)
    out_ref[...] = (
        jnp.dot(t, w3_ref[...], preferred_element_type=jnp.float32) / N
        + b3_ref[...][None, :]
    )


def kernel(features, edge_index, W1, b1, W2, b2, W3, b3):
    ei = edge_index.reshape(2, NW, NCH, C)
    eis = jnp.concatenate(
        [edge_index[0], jnp.full((EPAD - E,), N, jnp.int32)]
    ).reshape(NW, EPT2)
    eid = jnp.concatenate(
        [edge_index[1], jnp.arange(EPAD - E, dtype=jnp.int32)]
    ).reshape(NW, NCH2, 1, C2)

    degp = _deg_kernel(ei)

    norms, h1 = pl.pallas_call(
        _prep_body,
        out_shape=[
            jax.ShapeDtypeStruct((2, N), jnp.float32),
            jax.ShapeDtypeStruct((NP, H), jnp.float32),
        ],
    )(degp, features, W1)

    cpart = _cvec_kernel(ei, norms)

    agg1 = _agg_kernel(h1, eis, eid)

    h2 = pl.pallas_call(
        _epi_body,
        out_shape=jax.ShapeDtypeStruct((NP, H), jnp.float32),
    )(agg1, norms, b1, W2)

    agg2 = _agg_kernel(h2, eis, eid)

    pooled = pl.pallas_call(
        _fin_body,
        out_shape=jax.ShapeDtypeStruct((1, 1), jnp.float32),
    )(agg2, norms, b2, cpart, W3, b3)

    return pooled


# speedup vs baseline: 2.0995x; 1.8506x over previous
import dataclasses
import functools

import jax
import jax.numpy as jnp
from jax import lax
from jax.experimental import pallas as pl
from jax.experimental.pallas import tpu as pltpu
from jax.experimental.pallas import tpu_sc as plsc

N = 10000
E = 320000
H = 128

NC = 2
NS = 16
NW = NC * NS
EPT = E // NW
C = 80
NCH = EPT // C
RPT = 624
TAIL = N - NS * RPT

C2 = 128
NCH2 = 79
EPT2 = NCH2 * C2
EPAD = NW * EPT2
NP = N + 8

_MESH = plsc.VectorSubcoreMesh(
    core_axis_name="c", subcore_axis_name="s", num_cores=NC, num_subcores=NS
)

_SC_PARAMS = pltpu.CompilerParams()
if "needs_layout_passes" in pltpu.CompilerParams.__dataclass_fields__:
    _SC_PARAMS = dataclasses.replace(_SC_PARAMS, needs_layout_passes=False)


@functools.partial(
    pl.kernel,
    out_type=jax.ShapeDtypeStruct((NW, 2, N), jnp.float32),
    mesh=_MESH,
    scratch_types=[
        pltpu.VMEM((N,), jnp.float32),
        pltpu.VMEM((N,), jnp.float32),
        pltpu.VMEM((NCH, C), jnp.int32),
        pltpu.VMEM((NCH, C), jnp.int32),
    ],
    compiler_params=_SC_PARAMS,
)
def _deg_kernel(ei_hbm, out_hbm, dout_v, din_v, si_v, di_v):
    wid = lax.axis_index("s") * NC + lax.axis_index("c")

    pltpu.sync_copy(ei_hbm.at[0, wid], si_v)
    pltpu.sync_copy(ei_hbm.at[1, wid], di_v)

    @pl.loop(0, N, step=16)
    def _(i):
        z = jnp.zeros((16,), jnp.float32)
        dout_v[pl.ds(i, 16)] = z
        din_v[pl.ds(i, 16)] = z

    @pl.loop(0, NCH)
    def _(k):
        @pl.loop(0, C, step=16)
        def _(j):
            ones = jnp.ones((16,), jnp.float32)
            plsc.addupdate_scatter(dout_v, [si_v[k, pl.ds(j, 16)]], ones)
            plsc.addupdate_scatter(din_v, [di_v[k, pl.ds(j, 16)]], ones)

    pltpu.sync_copy(dout_v, out_hbm.at[wid, 0])
    pltpu.sync_copy(din_v, out_hbm.at[wid, 1])


@functools.partial(
    pl.kernel,
    out_type=jax.ShapeDtypeStruct((NW, N), jnp.float32),
    mesh=_MESH,
    scratch_types=[
        pltpu.VMEM((N,), jnp.float32),
        pltpu.VMEM((N,), jnp.float32),
        pltpu.VMEM((NCH, C), jnp.int32),
        pltpu.VMEM((NCH, C), jnp.int32),
    ],
    compiler_params=_SC_PARAMS,
)
def _cvec_kernel(ei_hbm, norms_hbm, out_hbm, nin_v, c_v, si_v, di_v):
    wid = lax.axis_index("s") * NC + lax.axis_index("c")
    pltpu.sync_copy(norms_hbm.at[1], nin_v)
    pltpu.sync_copy(ei_hbm.at[0, wid], si_v)
    pltpu.sync_copy(ei_hbm.at[1, wid], di_v)

    @pl.loop(0, N, step=16)
    def _(i):
        c_v[pl.ds(i, 16)] = jnp.zeros((16,), jnp.float32)

    @pl.loop(0, NCH)
    def _(k):
        @pl.loop(0, C, step=16)
        def _(j):
            vals = plsc.load_gather(nin_v, [di_v[k, pl.ds(j, 16)]])
            plsc.addupdate_scatter(c_v, [si_v[k, pl.ds(j, 16)]], vals)

    pltpu.sync_copy(c_v, out_hbm.at[wid])


@functools.partial(
    pl.kernel,
    out_type=jax.ShapeDtypeStruct((NC, N, H), jnp.float32),
    mesh=_MESH,
    scratch_types=[
        pltpu.VMEM_SHARED((N, H), jnp.float32),
        pltpu.VMEM((EPT2,), jnp.int32),
        pltpu.VMEM((1, C2), jnp.int32),
        pltpu.VMEM((1, C2), jnp.int32),
        pltpu.VMEM((C2, H), jnp.float32),
        pltpu.VMEM((C2, H), jnp.float32),
        pltpu.SemaphoreType.DMA,
        pltpu.SemaphoreType.DMA,
        pltpu.SemaphoreType.DMA,
        pltpu.SemaphoreType.DMA,
    ],
)
def _agg_kernel(h_hbm, eis_hbm, eid_hbm, out_hbm, acc_s, si_v, di_a, di_b,
                rows_a, rows_b, sem_ga, sem_gb, sem_ia, sem_ib):
    cid = lax.axis_index("c")
    sid = lax.axis_index("s")
    wid = sid * NC + cid

    pltpu.async_copy(eis_hbm.at[wid], si_v, sem_ga)

    @pl.loop(0, C2)
    def _(r):
        @pl.loop(0, H, step=16)
        def _(h):
            rows_b[r, pl.ds(h, 16)] = jnp.zeros((16,), jnp.float32)

    @pl.loop(0, 4 * C2, step=C2)
    def _(z):
        pltpu.sync_copy(rows_b, acc_s.at[pl.ds(sid * RPT + z, C2)])

    pltpu.sync_copy(
        rows_b.at[pl.ds(0, RPT - 4 * C2)],
        acc_s.at[pl.ds(sid * RPT + 4 * C2, RPT - 4 * C2)],
    )

    @pl.when(sid == NS - 1)
    def _():
        pltpu.sync_copy(rows_b.at[pl.ds(0, TAIL)], acc_s.at[pl.ds(NS * RPT, TAIL)])

    pltpu.make_async_copy(eis_hbm.at[wid], si_v, sem_ga).wait()
    pltpu.async_copy(h_hbm.at[si_v.at[pl.ds(0, C2)]], rows_a, sem_ga)
    pltpu.async_copy(eid_hbm.at[wid, 0], di_a, sem_ia)

    plsc.subcore_barrier()

    @pl.loop(0, NCH2 - 1, step=2)
    def _(k):
        pltpu.async_copy(h_hbm.at[si_v.at[pl.ds((k + 1) * C2, C2)]], rows_b, sem_gb)
        pltpu.async_copy(eid_hbm.at[wid, k + 1], di_b, sem_ib)
        pltpu.make_async_copy(h_hbm.at[si_v.at[pl.ds(k * C2, C2)]], rows_a, sem_ga).wait()
        pltpu.make_async_copy(eid_hbm.at[wid, k], di_a, sem_ia).wait()
        pltpu.sync_copy(rows_a, acc_s.at[di_a.at[0]], add=True)
        pltpu.async_copy(h_hbm.at[si_v.at[pl.ds((k + 2) * C2, C2)]], rows_a, sem_ga)
        pltpu.async_copy(eid_hbm.at[wid, k + 2], di_a, sem_ia)
        pltpu.make_async_copy(h_hbm.at[si_v.at[pl.ds((k + 1) * C2, C2)]], rows_b, sem_gb).wait()
        pltpu.make_async_copy(eid_hbm.at[wid, k + 1], di_b, sem_ib).wait()
        pltpu.sync_copy(rows_b, acc_s.at[di_b.at[0]], add=True)

    pltpu.make_async_copy(h_hbm.at[si_v.at[pl.ds((NCH2 - 1) * C2, C2)]], rows_a, sem_ga).wait()
    pltpu.make_async_copy(eid_hbm.at[wid, NCH2 - 1], di_a, sem_ia).wait()
    pltpu.sync_copy(rows_a, acc_s.at[di_a.at[0]], add=True)

    plsc.subcore_barrier()
    pltpu.sync_copy(
        acc_s.at[pl.ds(sid * RPT, RPT)], out_hbm.at[cid, pl.ds(sid * RPT, RPT)]
    )

    @pl.when(sid == NS - 1)
    def _():
        pltpu.sync_copy(
            acc_s.at[pl.ds(NS * RPT, TAIL)], out_hbm.at[cid, pl.ds(NS * RPT, TAIL)]
        )


def _prep_body(deg_ref, x_ref, w1_ref, norms_ref, h1_ref):
    deg = jnp.sum(deg_ref[...], axis=0)
    norms = jnp.where(deg > 0, lax.rsqrt(jnp.maximum(deg, 1.0)), 0.0)
    norms_ref[...] = norms
    h = x_ref[...] * norms[0][:, None]
    h1_ref[:N] = jnp.dot(h, w1_ref[...], preferred_element_type=jnp.float32)
    h1_ref[N:] = jnp.zeros((NP - N, H), jnp.float32)


def _epi_body(agg_ref, norms_ref, b_ref, w_ref, h_ref):
    ni = norms_ref[1][:, None]
    x = jnp.maximum(
        (agg_ref[0] + agg_ref[1]) * ni + b_ref[...][None, :], 0.0
    )
    h_ref[:N] = jnp.dot(
        x * norms_ref[0][:, None], w_ref[...], preferred_element_type=jnp.float32
    )
    h_ref[N:] = jnp.zeros((NP - N, H), jnp.float32)


def _fin_body(agg_ref, norms_ref, b2_ref, cp_ref, w3_ref, b3_ref, out_ref):
    ni = norms_ref[1][:, None]
    x3 = jnp.maximum(
        (agg_ref[0] + agg_ref[1]) * ni + b2_ref[...][None, :], 0.0
    )
    c = jnp.sum(cp_ref[...], axis=0)
    cn = (c * norms_ref[0])[None, :]
    t = jnp.dot(cn, x3, preferred_element_type=jnp.float32)
    out_ref[...] = (
        jnp.dot(t, w3_ref[...], preferred_element_type=jnp.float32) / N
        + b3_ref[...][None, :]
    )


def kernel(features, edge_index, W1, b1, W2, b2, W3, b3):
    ei = edge_index.reshape(2, NW, NCH, C)
    padt = EPT2 - EPT
    pad_s = N + jnp.arange(NW * padt, dtype=jnp.int32) % (NP - N)
    pad_d = jnp.arange(NW * padt, dtype=jnp.int32)
    eis = jnp.concatenate(
        [edge_index[0].reshape(NW, EPT), pad_s.reshape(NW, padt)], axis=1
    ).reshape(NW, EPT2)
    eid = jnp.concatenate(
        [edge_index[1].reshape(NW, EPT), pad_d.reshape(NW, padt)], axis=1
    ).reshape(NW, NCH2, 1, C2)

    degp = _deg_kernel(ei)

    norms, h1 = pl.pallas_call(
        _prep_body,
        out_shape=[
            jax.ShapeDtypeStruct((2, N), jnp.float32),
            jax.ShapeDtypeStruct((NP, H), jnp.float32),
        ],
    )(degp, features, W1)

    cpart = _cvec_kernel(ei, norms)

    agg1 = _agg_kernel(h1, eis, eid)

    h2 = pl.pallas_call(
        _epi_body,
        out_shape=jax.ShapeDtypeStruct((NP, H), jnp.float32),
    )(agg1, norms, b1, W2)

    agg2 = _agg_kernel(h2, eis, eid)

    pooled = pl.pallas_call(
        _fin_body,
        out_shape=jax.ShapeDtypeStruct((1, 1), jnp.float32),
    )(agg2, norms, b2, cpart, W3, b3)

    return pooled
